# baseline (speedup 1.0000x reference)
import functools

import jax
import jax.numpy as jnp
from jax import lax
from jax.experimental import pallas as pl
from jax.experimental.pallas import tpu as pltpu
from jax.experimental.pallas import tpu_sc as plsc

B = 4096
F = 512
NC = 8
NK = 5
HID = 64
OUT = 32
BLK = 256
INF = float("inf")

NQ = 4
QROWS = B // NQ
NWORK = 32
RPW = QROWS // NWORK
RB = 8
NCHUNK = RPW // RB
KPAD = 16


def _norms_body(x_ref, n2_ref):
    xf = x_ref[...]
    n2 = jnp.sum(xf * xf, axis=1, keepdims=True)
    n2_ref[...] = n2.reshape(1, B)


def _dist_feat_body(x_blk_ref, x_full_ref, n2_ref, cent_ref, temp_ref, cw_ref,
                    w1_ref, b1_ref, pd_ref, hpart_ref):
    xb = x_blk_ref[...]
    xf = x_full_ref[...]
    a2 = jnp.sum(xb * xb, axis=1, keepdims=True)
    ab = lax.dot_general(xb, xf, (((1,), (1,)), ((), ())),
                         preferred_element_type=jnp.float32)
    pd_ref[...] = a2 + n2_ref[...] - 2.0 * ab

    cent = cent_ref[...]
    c2 = jnp.sum(cent * cent, axis=1, keepdims=True)
    xc = lax.dot_general(xb, cent, (((1,), (1,)), ((), ())),
                         preferred_element_type=jnp.float32)
    dc2 = a2 + c2.reshape(1, NC) - 2.0 * xc
    dc = jnp.sqrt(jnp.maximum(dc2, 0.0) + 1e-12)
    z = -dc / temp_ref[0, 0]
    z = z - jnp.max(z, axis=1, keepdims=True)
    ez = jnp.exp(z)
    assign = ez / jnp.sum(ez, axis=1, keepdims=True) * cw_ref[...]

    mu = jnp.mean(xb, axis=1, keepdims=True)
    xc0 = xb - mu
    var = jnp.sum(xc0 * xc0, axis=1, keepdims=True) * (1.0 / (F - 1))
    lstd = jnp.sqrt(var) + 1e-8
    mx = jnp.max(xb, axis=1, keepdims=True)
    e = jnp.exp(xb - mx)
    s = jnp.sum(e, axis=1, keepdims=True)
    lse = mx + jnp.log(s)
    logp = xb - lse
    ent = -jnp.sum(jnp.exp(logp) * logp, axis=1, keepdims=True)

    w1 = w1_ref[...]
    h = b1_ref[...]
    for j in range(NC):
        h = h + assign[:, j:j + 1] * w1[j:j + 1, :]
    h = h + mu * w1[13:14, :] + lstd * w1[14:15, :] + ent * w1[15:16, :]
    hpart_ref[...] = h


def _make_topk_sc_body(row0):
    def _topk_sc_body(pd_hbm, out_hbm, rows0, rows1, out_v, gmin, mreg,
                      sem0, sem1):
        cid = lax.axis_index("c")
        sid = lax.axis_index("s")
        wid = sid * 2 + cid
        base = wid * RPW

        lane = lax.broadcasted_iota(jnp.int32, (16,), 0)
        inf16 = jnp.full((16,), INF, jnp.float32)
        one_lane = lane == 0
        bufs = (rows0, rows1)
        sems = (sem0, sem1)

        def start(ci, b):
            pltpu.async_copy(pd_hbm.at[pl.ds(base + ci * RB, RB)],
                             bufs[b], sems[b])

        def wait(ci, b):
            pltpu.make_async_copy(pd_hbm.at[pl.ds(base + ci * RB, RB)],
                                  bufs[b], sems[b]).wait()

        ninf16 = jnp.full((16,), -INF, jnp.float32)

        def process(ci, b):
            rows_v = bufs[b]
            for rl in range(RB):
                r_abs = row0 + base + ci * RB + rl
                rl16 = jnp.full((16,), rl, jnp.int32)
                c16 = jnp.full((16,), r_abs, jnp.int32)
                plsc.store_scatter(rows_v, [rl16, c16], inf16, mask=one_lane)

                def p1(i, m):
                    goff = i * 128
                    mg = rows_v[rl, pl.ds(goff, 16)]
                    for u in range(1, 8):
                        mg = jnp.minimum(mg, rows_v[rl, pl.ds(goff + u * 16, 16)])
                    gmin[pl.ds(i * 16, 16)] = mg
                    return jnp.minimum(m, mg)

                m = lax.fori_loop(0, B // 128, p1, inf16)

                sk, _ = plsc.sort_key_val(m, m)
                t = jnp.max(jnp.where(lane == 4, sk, ninf16))

                for qq in range(NK):
                    mreg[pl.ds(qq * 16, 16)] = inf16

                def p2(g, carry):
                    mg = gmin[pl.ds(g * 16, 16)]

                    @pl.when(jnp.any(mg <= t))
                    def _():
                        m0 = mreg[pl.ds(0, 16)]
                        m1 = mreg[pl.ds(16, 16)]
                        m2 = mreg[pl.ds(32, 16)]
                        m3 = mreg[pl.ds(48, 16)]
                        m4 = mreg[pl.ds(64, 16)]
                        goff = g * 128
                        for u in range(8):
                            v = rows_v[rl, pl.ds(goff + u * 16, 16)]
                            bb = jnp.maximum(m0, v)
                            m0 = jnp.minimum(m0, v)
                            bb, m1 = jnp.maximum(m1, bb), jnp.minimum(m1, bb)
                            bb, m2 = jnp.maximum(m2, bb), jnp.minimum(m2, bb)
                            bb, m3 = jnp.maximum(m3, bb), jnp.minimum(m3, bb)
                            m4 = jnp.minimum(m4, bb)
                        mreg[pl.ds(0, 16)] = m0
                        mreg[pl.ds(16, 16)] = m1
                        mreg[pl.ds(32, 16)] = m2
                        mreg[pl.ds(48, 16)] = m3
                        mreg[pl.ds(64, 16)] = m4
                    return carry

                lax.fori_loop(0, B // 128, p2, 0)
                m0 = mreg[pl.ds(0, 16)]
                m1 = mreg[pl.ds(16, 16)]
                m2 = mreg[pl.ds(32, 16)]
                m3 = mreg[pl.ds(48, 16)]
                m4 = mreg[pl.ds(64, 16)]

                out16 = inf16
                for k in range(NK):
                    g = jnp.min(m0)
                    out16 = jnp.where(lane == k, g, out16)
                    f = plsc.all_reduce_ffs(m0 == g)
                    sel = lane == f
                    m0 = jnp.where(sel, m1, m0)
                    m1 = jnp.where(sel, m2, m1)
                    m2 = jnp.where(sel, m3, m2)
                    m3 = jnp.where(sel, m4, m3)
                    m4 = jnp.where(sel, inf16, m4)
                out_v[ci * RB + rl, pl.ds(0, 16)] = out16

        start(0, 0)

        def outer(i, carry):
            g = i * 2
            start(g + 1, 1)
            wait(g, 0)
            process(g, 0)

            @pl.when(g + 2 < NCHUNK)
            def _():
                start(g + 2, 0)

            wait(g + 1, 1)
            process(g + 1, 1)
            return carry

        lax.fori_loop(0, NCHUNK // 2, outer, 0)
        pltpu.sync_copy(out_v, out_hbm.at[pl.ds(base, RPW)])

    return _topk_sc_body


def _mlp_body(hpart_ref, knn_ref, w1_ref, w2_ref, b2_ref, out_ref):
    knn2 = knn_ref[...]
    knn = jnp.sqrt(jnp.maximum(knn2, 0.0) + 1e-12)
    w1 = w1_ref[...]
    h = hpart_ref[...]
    for k in range(NK):
        h = h + knn[:, k:k + 1] * w1[NC + k:NC + k + 1, :]
    h = jnp.maximum(h, 0.0)
    out = lax.dot_general(h, w2_ref[...], (((1,), (0,)), ((), ())),
                          preferred_element_type=jnp.float32) + b2_ref[...]
    out_ref[...] = out


@jax.jit
def kernel(x, cluster_centers, temperature, cluster_weights, W1, b1, W2, b2):
    temp = temperature.reshape(1, 1)
    cw = cluster_weights.reshape(1, NC)
    b1r = b1.reshape(1, HID)
    b2r = b2.reshape(1, OUT)

    qgrid = QROWS // BLK
    mesh = plsc.VectorSubcoreMesh(core_axis_name="c", subcore_axis_name="s")

    n2 = pl.pallas_call(
        _norms_body,
        in_specs=[pl.BlockSpec((B, F), lambda: (0, 0))],
        out_specs=pl.BlockSpec((1, B), lambda: (0, 0)),
        out_shape=jax.ShapeDtypeStruct((1, B), jnp.float32),
    )(x)

    def stage_a(q):
        return pl.pallas_call(
            _dist_feat_body,
            grid=(qgrid,),
            in_specs=[
                pl.BlockSpec((BLK, F), lambda i, q=q: (q * qgrid + i, 0)),
                pl.BlockSpec((B, F), lambda i: (0, 0)),
                pl.BlockSpec((1, B), lambda i: (0, 0)),
                pl.BlockSpec((NC, F), lambda i: (0, 0)),
                pl.BlockSpec((1, 1), lambda i: (0, 0)),
                pl.BlockSpec((1, NC), lambda i: (0, 0)),
                pl.BlockSpec((NC + NK + 3, HID), lambda i: (0, 0)),
                pl.BlockSpec((1, HID), lambda i: (0, 0)),
            ],
            out_specs=[
                pl.BlockSpec((BLK, B), lambda i: (i, 0)),
                pl.BlockSpec((BLK, HID), lambda i: (i, 0)),
            ],
            out_shape=[
                jax.ShapeDtypeStruct((QROWS, B), jnp.float32),
                jax.ShapeDtypeStruct((QROWS, HID), jnp.float32),
            ],
        )(x, x, n2, cluster_centers, temp, cw, W1, b1r)

    def stage_b(q, pdq):
        topk = functools.partial(
            pl.kernel, mesh=mesh,
            out_type=jax.ShapeDtypeStruct((QROWS, KPAD), jnp.float32),
            scratch_types=[
                pltpu.VMEM((RB, B), jnp.float32),
                pltpu.VMEM((RB, B), jnp.float32),
                pltpu.VMEM((RPW, KPAD), jnp.float32),
                pltpu.VMEM(((B // 128) * 16,), jnp.float32),
                pltpu.VMEM((NK * 16,), jnp.float32),
                pltpu.SemaphoreType.DMA,
                pltpu.SemaphoreType.DMA,
            ],
            compiler_params=pltpu.CompilerParams(needs_layout_passes=False),
        )(_make_topk_sc_body(q * QROWS))
        return topk(pdq)

    def stage_c(hpq, knnq):
        return pl.pallas_call(
            _mlp_body,
            grid=(qgrid,),
            in_specs=[
                pl.BlockSpec((BLK, HID), lambda i: (i, 0)),
                pl.BlockSpec((BLK, KPAD), lambda i: (i, 0)),
                pl.BlockSpec((NC + NK + 3, HID), lambda i: (0, 0)),
                pl.BlockSpec((HID, OUT), lambda i: (0, 0)),
                pl.BlockSpec((1, OUT), lambda i: (0, 0)),
            ],
            out_specs=pl.BlockSpec((BLK, OUT), lambda i: (i, 0)),
            out_shape=jax.ShapeDtypeStruct((QROWS, OUT), jnp.float32),
        )(hpq, knnq, W1, W2, b2r)

    outs = []
    pds = []
    hps = []
    for q in range(NQ):
        pdq, hpq = stage_a(q)
        pds.append(pdq)
        hps.append(hpq)
    for q in range(NQ):
        knnq = stage_b(q, pds[q])
        outs.append(stage_c(hps[q], knnq))
    return jnp.concatenate(outs, axis=0)


# speedup vs baseline: 1.6333x; 1.6333x over previous
import functools

import jax
import jax.numpy as jnp
from jax import lax
from jax.experimental import pallas as pl
from jax.experimental.pallas import tpu as pltpu
from jax.experimental.pallas import tpu_sc as plsc

B = 4096
F = 512
NC = 8
NK = 5
HID = 64
OUT = 32
BLK = 256
INF = float("inf")

NQ = 4
QROWS = B // NQ
NWORK = 32
RPW = QROWS // NWORK
RB = 8
NCHUNK = RPW // RB
KPAD = 16


def _norms_body(x_ref, n2_ref):
    xf = x_ref[...]
    n2 = jnp.sum(xf * xf, axis=1, keepdims=True)
    n2_ref[...] = n2.reshape(1, B)


def _dist_feat_body(x_blk_ref, x_full_ref, n2_ref, cent_ref, temp_ref, cw_ref,
                    w1_ref, b1_ref, pd_ref, hpart_ref):
    xb = x_blk_ref[...]
    xf = x_full_ref[...]
    a2 = jnp.sum(xb * xb, axis=1, keepdims=True)
    ab = lax.dot_general(xb, xf, (((1,), (1,)), ((), ())),
                         preferred_element_type=jnp.float32)
    pd_ref[...] = a2 + n2_ref[...] - 2.0 * ab

    cent = cent_ref[...]
    c2 = jnp.sum(cent * cent, axis=1, keepdims=True)
    xc = lax.dot_general(xb, cent, (((1,), (1,)), ((), ())),
                         preferred_element_type=jnp.float32)
    dc2 = a2 + c2.reshape(1, NC) - 2.0 * xc
    dc = jnp.sqrt(jnp.maximum(dc2, 0.0) + 1e-12)
    z = -dc / temp_ref[0, 0]
    z = z - jnp.max(z, axis=1, keepdims=True)
    ez = jnp.exp(z)
    assign = ez / jnp.sum(ez, axis=1, keepdims=True) * cw_ref[...]

    mu = jnp.mean(xb, axis=1, keepdims=True)
    xc0 = xb - mu
    var = jnp.sum(xc0 * xc0, axis=1, keepdims=True) * (1.0 / (F - 1))
    lstd = jnp.sqrt(var) + 1e-8
    mx = jnp.max(xb, axis=1, keepdims=True)
    e = jnp.exp(xb - mx)
    s = jnp.sum(e, axis=1, keepdims=True)
    lse = mx + jnp.log(s)
    logp = xb - lse
    ent = -jnp.sum(jnp.exp(logp) * logp, axis=1, keepdims=True)

    w1 = w1_ref[...]
    h = b1_ref[...]
    for j in range(NC):
        h = h + assign[:, j:j + 1] * w1[j:j + 1, :]
    h = h + mu * w1[13:14, :] + lstd * w1[14:15, :] + ent * w1[15:16, :]
    hpart_ref[...] = h


def _make_topk_sc_body(row0):
    def _topk_sc_body(pd_hbm, out_hbm, rows0, rows1, out_v, sem0, sem1):
        cid = lax.axis_index("c")
        sid = lax.axis_index("s")
        wid = sid * 2 + cid
        base = wid * RPW

        lane = lax.broadcasted_iota(jnp.int32, (16,), 0)
        inf16 = jnp.full((16,), INF, jnp.float32)
        one_lane = lane == 0
        bufs = (rows0, rows1)
        sems = (sem0, sem1)

        def start(ci, b):
            pltpu.async_copy(pd_hbm.at[pl.ds(base + ci * RB, RB)],
                             bufs[b], sems[b])

        def wait(ci, b):
            pltpu.make_async_copy(pd_hbm.at[pl.ds(base + ci * RB, RB)],
                                  bufs[b], sems[b]).wait()

        def process(ci, b):
            rows_v = bufs[b]
            for rl in range(RB):
                r_abs = row0 + base + ci * RB + rl
                rl16 = jnp.full((16,), rl, jnp.int32)
                c16 = jnp.full((16,), r_abs, jnp.int32)
                plsc.store_scatter(rows_v, [rl16, c16], inf16, mask=one_lane)

                def scan_body(i, ms):
                    m0, m1, m2, m3, m4 = ms
                    off = i * 128
                    for u in range(8):
                        v = rows_v[rl, pl.ds(off + u * 16, 16)]
                        bb = jnp.maximum(m0, v)
                        m0 = jnp.minimum(m0, v)
                        bb, m1 = jnp.maximum(m1, bb), jnp.minimum(m1, bb)
                        bb, m2 = jnp.maximum(m2, bb), jnp.minimum(m2, bb)
                        bb, m3 = jnp.maximum(m3, bb), jnp.minimum(m3, bb)
                        m4 = jnp.minimum(m4, bb)
                    return (m0, m1, m2, m3, m4)

                m0, m1, m2, m3, m4 = lax.fori_loop(
                    0, B // 128, scan_body, (inf16, inf16, inf16, inf16, inf16))

                out16 = inf16
                for k in range(NK):
                    g = jnp.min(m0)
                    out16 = jnp.where(lane == k, g, out16)
                    f = plsc.all_reduce_ffs(m0 == g)
                    sel = lane == f
                    m0 = jnp.where(sel, m1, m0)
                    m1 = jnp.where(sel, m2, m1)
                    m2 = jnp.where(sel, m3, m2)
                    m3 = jnp.where(sel, m4, m3)
                    m4 = jnp.where(sel, inf16, m4)
                out_v[ci * RB + rl, pl.ds(0, 16)] = out16

        start(0, 0)

        def outer(i, carry):
            g = i * 2
            start(g + 1, 1)
            wait(g, 0)
            process(g, 0)

            @pl.when(g + 2 < NCHUNK)
            def _():
                start(g + 2, 0)

            wait(g + 1, 1)
            process(g + 1, 1)
            return carry

        lax.fori_loop(0, NCHUNK // 2, outer, 0)
        pltpu.sync_copy(out_v, out_hbm.at[pl.ds(base, RPW)])

    return _topk_sc_body


def _mlp_body(hpart_ref, knn_ref, w1_ref, w2_ref, b2_ref, out_ref):
    knn2 = knn_ref[...]
    knn = jnp.sqrt(jnp.maximum(knn2, 0.0) + 1e-12)
    w1 = w1_ref[...]
    h = hpart_ref[...]
    for k in range(NK):
        h = h + knn[:, k:k + 1] * w1[NC + k:NC + k + 1, :]
    h = jnp.maximum(h, 0.0)
    out = lax.dot_general(h, w2_ref[...], (((1,), (0,)), ((), ())),
                          preferred_element_type=jnp.float32) + b2_ref[...]
    out_ref[...] = out


@jax.jit
def kernel(x, cluster_centers, temperature, cluster_weights, W1, b1, W2, b2):
    temp = temperature.reshape(1, 1)
    cw = cluster_weights.reshape(1, NC)
    b1r = b1.reshape(1, HID)
    b2r = b2.reshape(1, OUT)

    qgrid = QROWS // BLK
    mesh = plsc.VectorSubcoreMesh(core_axis_name="c", subcore_axis_name="s")

    n2 = pl.pallas_call(
        _norms_body,
        in_specs=[pl.BlockSpec((B, F), lambda: (0, 0))],
        out_specs=pl.BlockSpec((1, B), lambda: (0, 0)),
        out_shape=jax.ShapeDtypeStruct((1, B), jnp.float32),
    )(x)

    def stage_a(q):
        return pl.pallas_call(
            _dist_feat_body,
            grid=(qgrid,),
            in_specs=[
                pl.BlockSpec((BLK, F), lambda i, q=q: (q * qgrid + i, 0)),
                pl.BlockSpec((B, F), lambda i: (0, 0)),
                pl.BlockSpec((1, B), lambda i: (0, 0)),
                pl.BlockSpec((NC, F), lambda i: (0, 0)),
                pl.BlockSpec((1, 1), lambda i: (0, 0)),
                pl.BlockSpec((1, NC), lambda i: (0, 0)),
                pl.BlockSpec((NC + NK + 3, HID), lambda i: (0, 0)),
                pl.BlockSpec((1, HID), lambda i: (0, 0)),
            ],
            out_specs=[
                pl.BlockSpec((BLK, B), lambda i: (i, 0)),
                pl.BlockSpec((BLK, HID), lambda i: (i, 0)),
            ],
            out_shape=[
                jax.ShapeDtypeStruct((QROWS, B), jnp.float32),
                jax.ShapeDtypeStruct((QROWS, HID), jnp.float32),
            ],
        )(x, x, n2, cluster_centers, temp, cw, W1, b1r)

    def stage_b(q, pdq):
        topk = functools.partial(
            pl.kernel, mesh=mesh,
            out_type=jax.ShapeDtypeStruct((QROWS, KPAD), jnp.float32),
            scratch_types=[
                pltpu.VMEM((RB, B), jnp.float32),
                pltpu.VMEM((RB, B), jnp.float32),
                pltpu.VMEM((RPW, KPAD), jnp.float32),
                pltpu.SemaphoreType.DMA,
                pltpu.SemaphoreType.DMA,
            ],
            compiler_params=pltpu.CompilerParams(needs_layout_passes=False),
        )(_make_topk_sc_body(q * QROWS))
        return topk(pdq)

    def stage_c(hpq, knnq):
        return pl.pallas_call(
            _mlp_body,
            grid=(qgrid,),
            in_specs=[
                pl.BlockSpec((BLK, HID), lambda i: (i, 0)),
                pl.BlockSpec((BLK, KPAD), lambda i: (i, 0)),
                pl.BlockSpec((NC + NK + 3, HID), lambda i: (0, 0)),
                pl.BlockSpec((HID, OUT), lambda i: (0, 0)),
                pl.BlockSpec((1, OUT), lambda i: (0, 0)),
            ],
            out_specs=pl.BlockSpec((BLK, OUT), lambda i: (i, 0)),
            out_shape=jax.ShapeDtypeStruct((QROWS, OUT), jnp.float32),
        )(hpq, knnq, W1, W2, b2r)

    outs = []
    pds = []
    hps = []
    for q in range(NQ):
        pdq, hpq = stage_a(q)
        pds.append(pdq)
        hps.append(hpq)
    for q in range(NQ):
        knnq = stage_b(q, pds[q])
        outs.append(stage_c(hps[q], knnq))
    return jnp.concatenate(outs, axis=0)


# speedup vs baseline: 1.6583x; 1.0153x over previous
import functools

import jax
import jax.numpy as jnp
from jax import lax
from jax.experimental import pallas as pl
from jax.experimental.pallas import tpu as pltpu
from jax.experimental.pallas import tpu_sc as plsc

B = 4096
F = 512
NC = 8
NK = 5
HID = 64
OUT = 32
BLK = 256
INF = float("inf")

NQ = 4
QROWS = B // NQ
NWORK = 32
RPW = QROWS // NWORK
RB = 8
NCHUNK = RPW // RB
KPAD = 16


def _norms_body(x_ref, n2_ref):
    xf = x_ref[...]
    n2 = jnp.sum(xf * xf, axis=1, keepdims=True)
    n2_ref[...] = n2.reshape(1, B)


def _dist_feat_body(x_blk_ref, xb16_ref, xf16_ref, n2_ref, cent_ref, temp_ref,
                    cw_ref, w1_ref, b1_ref, pd_ref, hpart_ref):
    xb = x_blk_ref[...]
    a2 = jnp.sum(xb * xb, axis=1, keepdims=True)
    ab = lax.dot_general(xb16_ref[...], xf16_ref[...], (((1,), (1,)), ((), ())),
                         preferred_element_type=jnp.float32)
    pd_ref[...] = a2 + n2_ref[...] - 2.0 * ab

    cent = cent_ref[...]
    c2 = jnp.sum(cent * cent, axis=1, keepdims=True)
    xc = lax.dot_general(xb, cent, (((1,), (1,)), ((), ())),
                         preferred_element_type=jnp.float32)
    dc2 = a2 + c2.reshape(1, NC) - 2.0 * xc
    dc = jnp.sqrt(jnp.maximum(dc2, 0.0) + 1e-12)
    z = -dc / temp_ref[0, 0]
    z = z - jnp.max(z, axis=1, keepdims=True)
    ez = jnp.exp(z)
    assign = ez / jnp.sum(ez, axis=1, keepdims=True) * cw_ref[...]

    mu = jnp.mean(xb, axis=1, keepdims=True)
    xc0 = xb - mu
    var = jnp.sum(xc0 * xc0, axis=1, keepdims=True) * (1.0 / (F - 1))
    lstd = jnp.sqrt(var) + 1e-8
    mx = jnp.max(xb, axis=1, keepdims=True)
    e = jnp.exp(xb - mx)
    s = jnp.sum(e, axis=1, keepdims=True)
    lse = mx + jnp.log(s)
    logp = xb - lse
    ent = -jnp.sum(jnp.exp(logp) * logp, axis=1, keepdims=True)

    w1 = w1_ref[...]
    h = b1_ref[...]
    for j in range(NC):
        h = h + assign[:, j:j + 1] * w1[j:j + 1, :]
    h = h + mu * w1[13:14, :] + lstd * w1[14:15, :] + ent * w1[15:16, :]
    hpart_ref[...] = h


def _make_topk_sc_body(row0):
    def _topk_sc_body(pd_hbm, out_hbm, rows0, rows1, out_v, sem0, sem1):
        cid = lax.axis_index("c")
        sid = lax.axis_index("s")
        wid = sid * 2 + cid
        base = wid * RPW

        lane = lax.broadcasted_iota(jnp.int32, (16,), 0)
        inf16 = jnp.full((16,), INF, jnp.float32)
        one_lane = lane == 0
        bufs = (rows0, rows1)
        sems = (sem0, sem1)

        def start(ci, b):
            pltpu.async_copy(pd_hbm.at[pl.ds(base + ci * RB, RB)],
                             bufs[b], sems[b])

        def wait(ci, b):
            pltpu.make_async_copy(pd_hbm.at[pl.ds(base + ci * RB, RB)],
                                  bufs[b], sems[b]).wait()

        def merge5(m0, m1, m2, m3, m4):
            out16 = inf16
            for k in range(NK):
                g = jnp.min(m0)
                out16 = jnp.where(lane == k, g, out16)
                f = plsc.all_reduce_ffs(m0 == g)
                sel = lane == f
                m0 = jnp.where(sel, m1, m0)
                m1 = jnp.where(sel, m2, m1)
                m2 = jnp.where(sel, m3, m2)
                m3 = jnp.where(sel, m4, m3)
                m4 = jnp.where(sel, inf16, m4)
            return out16

        def process(ci, b):
            rows_v = bufs[b]
            for rl in range(0, RB, 2):
                for dd in range(2):
                    r_abs = row0 + base + ci * RB + rl + dd
                    rl16 = jnp.full((16,), rl + dd, jnp.int32)
                    c16 = jnp.full((16,), r_abs, jnp.int32)
                    plsc.store_scatter(rows_v, [rl16, c16], inf16,
                                       mask=one_lane)

                def scan_body(i, ms):
                    a0, a1, a2, a3, a4, b0, b1, b2, b3, b4 = ms
                    off = i * 128
                    for u in range(8):
                        va = rows_v[rl, pl.ds(off + u * 16, 16)]
                        vb = rows_v[rl + 1, pl.ds(off + u * 16, 16)]
                        ta = jnp.maximum(a0, va)
                        a0 = jnp.minimum(a0, va)
                        tb = jnp.maximum(b0, vb)
                        b0 = jnp.minimum(b0, vb)
                        ta, a1 = jnp.maximum(a1, ta), jnp.minimum(a1, ta)
                        tb, b1 = jnp.maximum(b1, tb), jnp.minimum(b1, tb)
                        ta, a2 = jnp.maximum(a2, ta), jnp.minimum(a2, ta)
                        tb, b2 = jnp.maximum(b2, tb), jnp.minimum(b2, tb)
                        ta, a3 = jnp.maximum(a3, ta), jnp.minimum(a3, ta)
                        tb, b3 = jnp.maximum(b3, tb), jnp.minimum(b3, tb)
                        a4 = jnp.minimum(a4, ta)
                        b4 = jnp.minimum(b4, tb)
                    return (a0, a1, a2, a3, a4, b0, b1, b2, b3, b4)

                ms = lax.fori_loop(0, B // 128, scan_body, (inf16,) * 10)
                out_v[ci * RB + rl, pl.ds(0, 16)] = merge5(*ms[:5])
                out_v[ci * RB + rl + 1, pl.ds(0, 16)] = merge5(*ms[5:])

        start(0, 0)

        def outer(i, carry):
            g = i * 2
            start(g + 1, 1)
            wait(g, 0)
            process(g, 0)

            @pl.when(g + 2 < NCHUNK)
            def _():
                start(g + 2, 0)

            wait(g + 1, 1)
            process(g + 1, 1)
            return carry

        lax.fori_loop(0, NCHUNK // 2, outer, 0)
        pltpu.sync_copy(out_v, out_hbm.at[pl.ds(base, RPW)])

    return _topk_sc_body


def _mlp_body(hpart_ref, knn_ref, w1_ref, w2_ref, b2_ref, out_ref):
    knn2 = knn_ref[...]
    knn = jnp.sqrt(jnp.maximum(knn2, 0.0) + 1e-12)
    w1 = w1_ref[...]
    h = hpart_ref[...]
    for k in range(NK):
        h = h + knn[:, k:k + 1] * w1[NC + k:NC + k + 1, :]
    h = jnp.maximum(h, 0.0)
    out = lax.dot_general(h, w2_ref[...], (((1,), (0,)), ((), ())),
                          preferred_element_type=jnp.float32) + b2_ref[...]
    out_ref[...] = out


@jax.jit
def kernel(x, cluster_centers, temperature, cluster_weights, W1, b1, W2, b2):
    temp = temperature.reshape(1, 1)
    cw = cluster_weights.reshape(1, NC)
    b1r = b1.reshape(1, HID)
    b2r = b2.reshape(1, OUT)

    qgrid = QROWS // BLK
    mesh = plsc.VectorSubcoreMesh(core_axis_name="c", subcore_axis_name="s")
    x16 = x.astype(jnp.bfloat16)

    n2 = pl.pallas_call(
        _norms_body,
        in_specs=[pl.BlockSpec((B, F), lambda: (0, 0))],
        out_specs=pl.BlockSpec((1, B), lambda: (0, 0)),
        out_shape=jax.ShapeDtypeStruct((1, B), jnp.float32),
    )(x)

    def stage_a(q):
        return pl.pallas_call(
            _dist_feat_body,
            grid=(qgrid,),
            in_specs=[
                pl.BlockSpec((BLK, F), lambda i, q=q: (q * qgrid + i, 0)),
                pl.BlockSpec((BLK, F), lambda i, q=q: (q * qgrid + i, 0)),
                pl.BlockSpec((B, F), lambda i: (0, 0)),
                pl.BlockSpec((1, B), lambda i: (0, 0)),
                pl.BlockSpec((NC, F), lambda i: (0, 0)),
                pl.BlockSpec((1, 1), lambda i: (0, 0)),
                pl.BlockSpec((1, NC), lambda i: (0, 0)),
                pl.BlockSpec((NC + NK + 3, HID), lambda i: (0, 0)),
                pl.BlockSpec((1, HID), lambda i: (0, 0)),
            ],
            out_specs=[
                pl.BlockSpec((BLK, B), lambda i: (i, 0)),
                pl.BlockSpec((BLK, HID), lambda i: (i, 0)),
            ],
            out_shape=[
                jax.ShapeDtypeStruct((QROWS, B), jnp.float32),
                jax.ShapeDtypeStruct((QROWS, HID), jnp.float32),
            ],
        )(x, x16, x16, n2, cluster_centers, temp, cw, W1, b1r)

    def stage_b(q, pdq):
        topk = functools.partial(
            pl.kernel, mesh=mesh,
            out_type=jax.ShapeDtypeStruct((QROWS, KPAD), jnp.float32),
            scratch_types=[
                pltpu.VMEM((RB, B), jnp.float32),
                pltpu.VMEM((RB, B), jnp.float32),
                pltpu.VMEM((RPW, KPAD), jnp.float32),
                pltpu.SemaphoreType.DMA,
                pltpu.SemaphoreType.DMA,
            ],
            compiler_params=pltpu.CompilerParams(needs_layout_passes=False),
        )(_make_topk_sc_body(q * QROWS))
        return topk(pdq)

    def stage_c(hpq, knnq):
        return pl.pallas_call(
            _mlp_body,
            grid=(qgrid,),
            in_specs=[
                pl.BlockSpec((BLK, HID), lambda i: (i, 0)),
                pl.BlockSpec((BLK, KPAD), lambda i: (i, 0)),
                pl.BlockSpec((NC + NK + 3, HID), lambda i: (0, 0)),
                pl.BlockSpec((HID, OUT), lambda i: (0, 0)),
                pl.BlockSpec((1, OUT), lambda i: (0, 0)),
            ],
            out_specs=pl.BlockSpec((BLK, OUT), lambda i: (i, 0)),
            out_shape=jax.ShapeDtypeStruct((QROWS, OUT), jnp.float32),
        )(hpq, knnq, W1, W2, b2r)

    outs = []
    pds = []
    hps = []
    for q in range(NQ):
        pdq, hpq = stage_a(q)
        pds.append(pdq)
        hps.append(hpq)
    for q in range(NQ):
        knnq = stage_b(q, pds[q])
        outs.append(stage_c(hps[q], knnq))
    return jnp.concatenate(outs, axis=0)
